# baseline (speedup 1.0000x reference)
import functools

import jax
import jax.numpy as jnp
from jax import lax
from jax.experimental import pallas as pl
from jax.experimental.pallas import tpu as pltpu
from jax.experimental.pallas import tpu_sc as plsc

_CHUNK = 64


def _build_gather(B, V, D):
    info = plsc.get_sparse_core_info()
    NC, NS, L = info.num_cores, info.num_subcores, info.num_lanes
    NW = NC * NS
    assert B % NW == 0 and D == 64 and L == 16
    b_per_w = B // NW
    n_chunks = b_per_w // _CHUNK
    assert b_per_w % _CHUNK == 0 and _CHUNK % L == 0
    mesh = plsc.VectorSubcoreMesh(core_axis_name="c", subcore_axis_name="s")

    @functools.partial(
        pl.kernel,
        mesh=mesh,
        out_type=jax.ShapeDtypeStruct((D, B), jnp.float32),
        scratch_types=[
            pltpu.VMEM((b_per_w,), jnp.int32),
            pltpu.VMEM((2, 8, 8, 8 * _CHUNK), jnp.float32),
            pltpu.VMEM((D, b_per_w), jnp.float32),
            pltpu.SemaphoreType.DMA,
            pltpu.SemaphoreType.DMA,
        ],
        compiler_params=pltpu.CompilerParams(needs_layout_passes=False),
    )
    def gather_kernel(idx_hbm, tbl_hbm, out_hbm, idx_v, buf, slab, sem0, sem1):
        wid = lax.axis_index("s") * NC + lax.axis_index("c")
        base = wid * b_per_w

        pltpu.sync_copy(idx_hbm.at[pl.ds(base, b_per_w)], idx_v)

        iota = lax.iota(jnp.int32, L)
        seven = jnp.full((L,), 7, jnp.int32)
        eight = jnp.full((L,), 8, jnp.int32)
        iota8 = lax.mul(iota, eight)
        rcs = [jnp.full((L,), r, jnp.int32) for r in range(8)]
        scs = [jnp.full((L,), s, jnp.int32) for s in range(8)]
        sems = [sem0, sem1]

        def fire(c, slot):
            cbase = lax.mul(c, jnp.int32(_CHUNK))

            def fk(k, carry):
                g = lax.add(cbase, lax.mul(k, jnp.int32(L)))
                ivec = idx_v[pl.ds(g, L)]
                l0v = lax.mul(lax.div(ivec, eight), eight)
                for j in range(L):
                    l0 = pl.multiple_of(l0v[j], 8)
                    kb = lax.mul(
                        lax.add(lax.mul(k, jnp.int32(L)), jnp.int32(j)),
                        jnp.int32(8),
                    )
                    pltpu.async_copy(
                        tbl_hbm.at[:, :, pl.ds(l0, 8)],
                        buf.at[slot, :, :, pl.ds(kb, 8)],
                        sems[slot],
                    )
                return carry

            lax.fori_loop(0, _CHUNK // L, fk, 0)

        def wait_chunk(slot):
            pltpu.make_async_copy(
                tbl_hbm.at[:, :, pl.ds(0, 8 * _CHUNK)],
                buf.at[slot],
                sems[slot],
            ).wait()

        def extract(c, slot):
            cbase = lax.mul(c, jnp.int32(_CHUNK))

            def ek(k, carry):
                g = lax.add(cbase, lax.mul(k, jnp.int32(L)))
                ivec = idx_v[pl.ds(g, L)]
                lv = lax.add(
                    lax.add(
                        lax.broadcast(lax.mul(k, jnp.int32(8 * L)), (L,)),
                        iota8,
                    ),
                    lax.bitwise_and(ivec, seven),
                )
                for r in range(8):
                    for s in range(8):
                        v = plsc.load_gather(
                            buf.at[slot], [rcs[r], scs[s], lv]
                        )
                        slab[8 * r + s, pl.ds(g, L)] = v
                return carry

            lax.fori_loop(0, _CHUNK // L, ek, 0)

        T = n_chunks // 2
        zero = jnp.int32(0)
        fire(zero, 0)

        def pair_body(t, carry):
            c0 = lax.mul(t, jnp.int32(2))
            fire(lax.add(c0, jnp.int32(1)), 1)
            wait_chunk(0)
            extract(c0, 0)
            fire(lax.add(c0, jnp.int32(2)), 0)
            wait_chunk(1)
            extract(lax.add(c0, jnp.int32(1)), 1)
            return carry

        lax.fori_loop(0, T - 1, pair_body, 0)

        c0 = jnp.int32(2 * (T - 1))
        fire(lax.add(c0, jnp.int32(1)), 1)
        wait_chunk(0)
        extract(c0, 0)
        wait_chunk(1)
        extract(lax.add(c0, jnp.int32(1)), 1)

        pltpu.sync_copy(slab, out_hbm.at[:, pl.ds(base, b_per_w)])

    return gather_kernel


def kernel(state, embedding_table):
    B = state.shape[0]
    V, D = embedding_table.shape
    gather = _build_gather(B, V, D)
    tbl_t = embedding_table.T.reshape(D // 8, 8, V)
    out_t = gather(state.astype(jnp.int32), tbl_t)
    return out_t.T


# speedup vs baseline: 1.0145x; 1.0145x over previous
import functools

import jax
import jax.numpy as jnp
from jax import lax
from jax.experimental import pallas as pl
from jax.experimental.pallas import tpu as pltpu
from jax.experimental.pallas import tpu_sc as plsc

_CHUNK = 64


def _build_gather(B, V, D):
    info = plsc.get_sparse_core_info()
    NC, NS, L = info.num_cores, info.num_subcores, info.num_lanes
    NW = NC * NS
    assert B % NW == 0 and D == 64 and L == 16
    b_per_w = B // NW
    n_chunks = b_per_w // _CHUNK
    assert b_per_w % _CHUNK == 0 and _CHUNK % L == 0
    mesh = plsc.VectorSubcoreMesh(core_axis_name="c", subcore_axis_name="s")

    @functools.partial(
        pl.kernel,
        mesh=mesh,
        out_type=jax.ShapeDtypeStruct((D, B), jnp.float32),
        scratch_types=[
            pltpu.VMEM((b_per_w,), jnp.int32),
            pltpu.VMEM((2, 8, 8, 8 * _CHUNK), jnp.float32),
            pltpu.VMEM((D, b_per_w), jnp.float32),
            pltpu.SemaphoreType.DMA,
            pltpu.SemaphoreType.DMA,
            pltpu.SemaphoreType.DMA,
        ],
        compiler_params=pltpu.CompilerParams(needs_layout_passes=False),
    )
    def gather_kernel(
        idx_hbm, tbl_hbm, out_hbm, idx_v, buf, slab, sem0, sem1, osem
    ):
        wid = lax.axis_index("s") * NC + lax.axis_index("c")
        base = wid * b_per_w

        pltpu.sync_copy(idx_hbm.at[pl.ds(base, b_per_w)], idx_v)

        iota = lax.iota(jnp.int32, L)
        seven = jnp.full((L,), 7, jnp.int32)
        eight = jnp.full((L,), 8, jnp.int32)
        iota8 = lax.mul(iota, eight)
        rcs = [jnp.full((L,), r, jnp.int32) for r in range(8)]
        scs = [jnp.full((L,), s, jnp.int32) for s in range(8)]
        sems = [sem0, sem1]

        def fire(c, slot):
            cbase = lax.mul(c, jnp.int32(_CHUNK))

            def fk(k, carry):
                g = lax.add(cbase, lax.mul(k, jnp.int32(L)))
                ivec = idx_v[pl.ds(g, L)]
                l0v = lax.mul(lax.div(ivec, eight), eight)
                for j in range(L):
                    l0 = pl.multiple_of(l0v[j], 8)
                    kb = lax.mul(
                        lax.add(lax.mul(k, jnp.int32(L)), jnp.int32(j)),
                        jnp.int32(8),
                    )
                    pltpu.async_copy(
                        tbl_hbm.at[:, :, pl.ds(l0, 8)],
                        buf.at[slot, :, :, pl.ds(kb, 8)],
                        sems[slot],
                    )
                return carry

            lax.fori_loop(0, _CHUNK // L, fk, 0)

        def wait_chunk(slot):
            pltpu.make_async_copy(
                tbl_hbm.at[:, :, pl.ds(0, 8 * _CHUNK)],
                buf.at[slot],
                sems[slot],
            ).wait()

        def extract(c, slot):
            cbase = lax.mul(c, jnp.int32(_CHUNK))

            def ek(k, carry):
                g = lax.add(cbase, lax.mul(k, jnp.int32(L)))
                ivec = idx_v[pl.ds(g, L)]
                lv = lax.add(
                    lax.add(
                        lax.broadcast(lax.mul(k, jnp.int32(8 * L)), (L,)),
                        iota8,
                    ),
                    lax.bitwise_and(ivec, seven),
                )
                for r in range(8):
                    for s in range(8):
                        v = plsc.load_gather(
                            buf.at[slot], [rcs[r], scs[s], lv]
                        )
                        slab[8 * r + s, pl.ds(g, L)] = v
                return carry

            lax.fori_loop(0, _CHUNK // L, ek, 0)

        T = n_chunks // 2
        zero = jnp.int32(0)
        fire(zero, 0)

        def write_pair(c0):
            pb = lax.mul(c0, jnp.int32(_CHUNK))
            pltpu.async_copy(
                slab.at[:, pl.ds(pb, 2 * _CHUNK)],
                out_hbm.at[:, pl.ds(lax.add(jnp.int32(base), pb), 2 * _CHUNK)],
                osem,
            )

        def pair_body(t, carry):
            c0 = lax.mul(t, jnp.int32(2))
            fire(lax.add(c0, jnp.int32(1)), 1)
            wait_chunk(0)
            extract(c0, 0)
            fire(lax.add(c0, jnp.int32(2)), 0)
            wait_chunk(1)
            extract(lax.add(c0, jnp.int32(1)), 1)
            write_pair(c0)
            return carry

        lax.fori_loop(0, T - 1, pair_body, 0)

        c0 = jnp.int32(2 * (T - 1))
        fire(lax.add(c0, jnp.int32(1)), 1)
        wait_chunk(0)
        extract(c0, 0)
        wait_chunk(1)
        extract(lax.add(c0, jnp.int32(1)), 1)
        write_pair(c0)

        def wo(t, carry):
            pltpu.make_async_copy(
                slab.at[:, pl.ds(0, 2 * _CHUNK)],
                out_hbm.at[:, pl.ds(base, 2 * _CHUNK)],
                osem,
            ).wait()
            return carry

        lax.fori_loop(0, T, wo, 0)

    return gather_kernel


def kernel(state, embedding_table):
    B = state.shape[0]
    V, D = embedding_table.shape
    gather = _build_gather(B, V, D)
    tbl_t = embedding_table.T.reshape(D // 8, 8, V)
    out_t = gather(state.astype(jnp.int32), tbl_t)
    return out_t.T
